# baseline (speedup 1.0000x reference)
import functools

import jax
import jax.numpy as jnp
from jax import lax
from jax.experimental import pallas as pl
from jax.experimental.pallas import tpu as pltpu
from jax.experimental.pallas import tpu_sc as plsc

N = 10000
E = 320000
D = 128
L = 4

NC = 2
NS = 16
NW = NC * NS
EPT = E // NW
CH = 80
NCHUNK = EPT // CH
RPT0 = 632
RPTL = N - (NS - 1) * RPT0
DW = 16


def _tile_rows_copy(s, src_fn, dst_fn):
    r0 = pl.multiple_of(s * RPT0, 8)

    @pl.when(s < NS - 1)
    def _():
        pltpu.sync_copy(src_fn(r0, RPT0), dst_fn(r0, RPT0))

    @pl.when(s == NS - 1)
    def _():
        pltpu.sync_copy(src_fn(r0, RPTL), dst_fn(r0, RPTL))

NBD = 5
DGROUPS = NCHUNK // NBD


def _deg_body(src2_hbm, dst2_hbm, ones_hbm, zdeg_hbm, out_hbm,
              sidx, didx, ones_v, acc_s, acc_d, *sems):
    ssems = sems[:NBD]
    dsems = sems[NBD:]
    c = lax.axis_index("c")
    s = lax.axis_index("s")
    wid = c * NS + s
    cbase = wid * NCHUNK
    _tile_rows_copy(s, lambda r, n: zdeg_hbm.at[pl.ds(r, n)],
                    lambda r, n: acc_s.at[pl.ds(r, n)])
    _tile_rows_copy(s, lambda r, n: zdeg_hbm.at[pl.ds(r, n)],
                    lambda r, n: acc_d.at[pl.ds(r, n)])
    pltpu.sync_copy(src2_hbm.at[pl.ds(cbase, NCHUNK)], sidx)
    pltpu.sync_copy(dst2_hbm.at[pl.ds(cbase, NCHUNK)], didx)
    pltpu.sync_copy(ones_hbm, ones_v)
    plsc.subcore_barrier()

    def group(gi, carry):
        for bb in range(NBD):
            j = gi * NBD + bb

            @pl.when(j >= NBD)
            def _():
                pltpu.make_async_copy(ones_v, acc_s.at[sidx.at[j - NBD]],
                                      ssems[bb]).wait()
                pltpu.make_async_copy(ones_v, acc_d.at[didx.at[j - NBD]],
                                      dsems[bb]).wait()

            pltpu.async_copy(ones_v, acc_s.at[sidx.at[j]], ssems[bb],
                             add=True)
            pltpu.async_copy(ones_v, acc_d.at[didx.at[j]], dsems[bb],
                             add=True)
        return carry

    lax.fori_loop(0, DGROUPS, group, 0)
    for cc in range(NCHUNK - NBD, NCHUNK):
        b = cc % NBD
        pltpu.make_async_copy(ones_v, acc_s.at[sidx.at[cc]], ssems[b]).wait()
        pltpu.make_async_copy(ones_v, acc_d.at[didx.at[cc]], dsems[b]).wait()
    plsc.subcore_barrier()
    _tile_rows_copy(s, lambda r, n: acc_s.at[pl.ds(r, n)],
                    lambda r, n: out_hbm.at[c, 0, pl.ds(r, n)])
    _tile_rows_copy(s, lambda r, n: acc_d.at[pl.ds(r, n)],
                    lambda r, n: out_hbm.at[c, 1, pl.ds(r, n)])


NB = 4
KLAG = 3
AGROUPS = -(-(NCHUNK + KLAG) // NB)


def _agg_body(src2_hbm, dst2_hbm, m_hbm, zeros_hbm, out_hbm,
              sidx, didx, rows, acc, *sems):
    gsems = sems[:NB]
    ssems = sems[NB:]
    c = lax.axis_index("c")
    s = lax.axis_index("s")
    wid = c * NS + s
    cbase = wid * NCHUNK
    _tile_rows_copy(s, lambda r, n: zeros_hbm.at[pl.ds(r, n)],
                    lambda r, n: acc.at[pl.ds(r, n)])
    pltpu.sync_copy(src2_hbm.at[pl.ds(cbase, NCHUNK)], sidx)
    pltpu.sync_copy(dst2_hbm.at[pl.ds(cbase, NCHUNK)], didx)
    plsc.subcore_barrier()

    def group(gi, carry):
        for bb in range(NB):
            j = gi * NB + bb

            @pl.when(j < NCHUNK)
            def _():
                @pl.when(j >= NB)
                def _():
                    pltpu.make_async_copy(rows.at[bb],
                                          acc.at[didx.at[j - NB]],
                                          ssems[bb]).wait()

                pltpu.async_copy(m_hbm.at[sidx.at[j]], rows.at[bb],
                                 gsems[bb])

            @pl.when(jnp.logical_and(j >= KLAG, j < NCHUNK + KLAG))
            def _():
                cs = j - KLAG
                bs = (bb - KLAG) % NB
                pltpu.make_async_copy(m_hbm.at[sidx.at[cs]], rows.at[bs],
                                      gsems[bs]).wait()
                pltpu.async_copy(rows.at[bs], acc.at[didx.at[cs]],
                                 ssems[bs], add=True)
        return carry

    lax.fori_loop(0, AGROUPS, group, 0)
    for cc in range(NCHUNK - NB, NCHUNK):
        b = cc % NB
        pltpu.make_async_copy(rows.at[b], acc.at[didx.at[cc]],
                              ssems[b]).wait()
    plsc.subcore_barrier()
    _tile_rows_copy(s, lambda r, n: acc.at[pl.ds(r, n)],
                    lambda r, n: out_hbm.at[c, pl.ds(r, n)])


@functools.cache
def _sc_kernels():
    mesh = plsc.VectorSubcoreMesh(core_axis_name="c", subcore_axis_name="s")
    deg = pl.kernel(
        _deg_body,
        out_type=jax.ShapeDtypeStruct((NC, 2, N, DW), jnp.float32),
        mesh=mesh,
        compiler_params=pltpu.CompilerParams(use_tc_tiling_on_sc=False),
        scratch_types=[
            pltpu.VMEM((NCHUNK, CH), jnp.int32),
            pltpu.VMEM((NCHUNK, CH), jnp.int32),
            pltpu.VMEM((CH, DW), jnp.float32),
            pltpu.VMEM_SHARED((N, DW), jnp.float32),
            pltpu.VMEM_SHARED((N, DW), jnp.float32),
        ] + [pltpu.SemaphoreType.DMA] * (2 * NBD),
    )
    agg = pl.kernel(
        _agg_body,
        out_type=jax.ShapeDtypeStruct((NC, N, D // 2), jnp.int32),
        mesh=mesh,
        compiler_params=pltpu.CompilerParams(use_tc_tiling_on_sc=False),
        scratch_types=[
            pltpu.VMEM((NCHUNK, CH), jnp.int32),
            pltpu.VMEM((NCHUNK, CH), jnp.int32),
            pltpu.VMEM((NB, CH, D // 2), jnp.int32),
            pltpu.VMEM_SHARED((N, D // 2), jnp.int32),
        ] + [pltpu.SemaphoreType.DMA] * (2 * NB),
    )
    return deg, agg


N2 = N // 2
D2 = 2 * D
QBUDGET = 32766.0


def _bc2(v2, w):
    return jnp.concatenate([jnp.broadcast_to(v2[:, 0:1], (N2, w)),
                            jnp.broadcast_to(v2[:, 1:2], (N2, w))], axis=1)


def _blockdiag(w):
    z = jnp.zeros((D, D), dtype=w.dtype)
    return jnp.concatenate([jnp.concatenate([w, z], axis=1),
                            jnp.concatenate([z, w], axis=1)], axis=0)


def _quantize5(m5, dmax):
    amp = jnp.floor(QBUDGET / jnp.maximum(dmax, 1.0))
    mx = jnp.max(jnp.abs(m5), axis=0)
    mm = jnp.maximum(mx[:D], mx[D:])[None, :]
    inv_scale = jnp.maximum(mm, 1e-30) / (amp - 1.0)
    rs = 1.0 / jnp.concatenate([inv_scale, inv_scale], axis=1)
    q = jnp.round(m5 * rs).astype(jnp.int32)
    ai = amp.astype(jnp.int32)
    packed = jnp.concatenate(
        [(q[:, 64:128] << 16) + q[:, 0:64] + ai,
         (q[:, 192:256] << 16) + q[:, 128:192] + ai], axis=1)
    return packed, inv_scale, amp


def _embed_body(h_ref, w_ref, b_ref, deg5_ref, x_ref, mq_ref, dis_ref,
                dm_ref, qs_ref, dn_ref):
    w2 = _blockdiag(w_ref[...])
    b2 = jnp.concatenate([b_ref[...], b_ref[...]])
    x5 = jnp.dot(h_ref[...], w2, preferred_element_type=jnp.float32) + b2
    de = deg5_ref[...]
    dis = jnp.where(de > 0.0,
                    1.0 / jnp.sqrt(jnp.maximum(de, 1.0)),
                    0.0)
    dmax = jnp.max(de[1])
    m5 = x5 * _bc2(dis[0], D)
    mq, qs, _ = _quantize5(m5, dmax)
    x_ref[...] = x5
    mq_ref[...] = mq
    dis_ref[...] = dis
    dm_ref[...] = jnp.reshape(dmax, (1, 1))
    qs_ref[...] = qs
    dn_ref[...] = de[1]


_embed_call = pl.pallas_call(
    _embed_body,
    compiler_params=pltpu.CompilerParams(vmem_limit_bytes=100 * 1024 * 1024),
    out_shape=[
        jax.ShapeDtypeStruct((N2, D2), jnp.float32),
        jax.ShapeDtypeStruct((N2, D), jnp.int32),
        jax.ShapeDtypeStruct((2, N2, 2), jnp.float32),
        jax.ShapeDtypeStruct((1, 1), jnp.float32),
        jax.ShapeDtypeStruct((1, D), jnp.float32),
        jax.ShapeDtypeStruct((N2, 2), jnp.float32),
    ],
)


def _layer_body(part_ref, x_ref, dis_ref, dm_ref, qs_ref, dn_ref,
                w_ref, b_ref, g_ref, be_ref, y_ref, mq_ref, qs2_ref):
    dmax = dm_ref[0, 0]
    amp = jnp.floor(QBUDGET / jnp.maximum(dmax, 1.0))
    psum = part_ref[0] + part_ref[1]
    low = (psum & 0xFFFF).astype(jnp.float32) - _bc2(dn_ref[...], 64) * amp
    high = (psum >> 16).astype(jnp.float32)
    agg = jnp.concatenate([low[:, 0:64], high[:, 0:64],
                           low[:, 64:128], high[:, 64:128]], axis=1)
    qs = qs_ref[...]
    agg = agg * jnp.concatenate([qs, qs], axis=1) * _bc2(dis_ref[1], D)
    w2 = _blockdiag(w_ref[...])
    out = jnp.dot(agg, w2, preferred_element_type=jnp.float32)
    out = out + jnp.concatenate([b_ref[...], b_ref[...]])
    s1 = jnp.sum(out, axis=0)
    mean = (s1[:D] + s1[D:]) * (1.0 / N)
    cent = out - jnp.concatenate([mean, mean])
    s2 = jnp.sum(cent * cent, axis=0)
    var = (s2[:D] + s2[D:]) * (1.0 / N)
    rstd = lax.rsqrt(var + 1e-5) * g_ref[...]
    out = cent * jnp.concatenate([rstd, rstd]) \
        + jnp.concatenate([be_ref[...], be_ref[...]])
    y5 = jnp.maximum(out, 0.0) + x_ref[...]
    m5 = y5 * _bc2(dis_ref[0], D)
    mq, qs2, _ = _quantize5(m5, dmax)
    y_ref[...] = y5
    mq_ref[...] = mq
    qs2_ref[...] = qs2


_layer_call = pl.pallas_call(
    _layer_body,
    compiler_params=pltpu.CompilerParams(vmem_limit_bytes=100 * 1024 * 1024),
    out_shape=[
        jax.ShapeDtypeStruct((N2, D2), jnp.float32),
        jax.ShapeDtypeStruct((N2, D), jnp.int32),
        jax.ShapeDtypeStruct((1, D), jnp.float32),
    ],
)


def kernel(g, h, e, W_embed, b_embed, Ws, bs, gammas, betas):
    src2 = g[0].reshape(E // CH, CH)
    dst2 = g[1].reshape(E // CH, CH)
    ones = jnp.ones((CH, DW), dtype=jnp.float32)
    zdeg = jnp.zeros((N, DW), dtype=jnp.float32)
    zeros = jnp.zeros((N, D // 2), dtype=jnp.int32)
    deg_kernel, agg_kernel = _sc_kernels()
    degs = deg_kernel(src2, dst2, ones, zdeg)
    dsum = degs.reshape(NC, 2, N * DW // 128, 128)
    dsum = dsum[0] + dsum[1]
    deg5 = dsum.reshape(2, N, DW)[:, :, 0].reshape(2, N2, 2)
    x, mq, dis, dmax, qs, degn = _embed_call(h.reshape(N2, D2), W_embed,
                                             b_embed, deg5)
    for i in range(L):
        part = agg_kernel(src2, dst2, mq.reshape(N, D // 2), zeros)
        x, mq, qs = _layer_call(part.reshape(NC, N2, D), x, dis, dmax,
                                qs, degn, Ws[i], bs[i], gammas[i], betas[i])
    return x.reshape(N, D)


# speedup vs baseline: 1.0389x; 1.0389x over previous
import functools

import jax
import jax.numpy as jnp
from jax import lax
from jax.experimental import pallas as pl
from jax.experimental.pallas import tpu as pltpu
from jax.experimental.pallas import tpu_sc as plsc

N = 10000
E = 320000
D = 128
L = 4

NC = 2
NS = 16
NW = NC * NS
EPT = E // NW
CH = 80
NCHUNK = EPT // CH
RPT0 = 632
RPTL = N - (NS - 1) * RPT0
DW = 16


def _tile_rows_copy(s, src_fn, dst_fn):
    r0 = pl.multiple_of(s * RPT0, 8)

    @pl.when(s < NS - 1)
    def _():
        pltpu.sync_copy(src_fn(r0, RPT0), dst_fn(r0, RPT0))

    @pl.when(s == NS - 1)
    def _():
        pltpu.sync_copy(src_fn(r0, RPTL), dst_fn(r0, RPTL))

NBD = 5
DGROUPS = NCHUNK // NBD


def _deg_body(src2_hbm, dst2_hbm, ones_hbm, zdeg_hbm, out_hbm,
              sidx, didx, ones_v, acc_s, acc_d, *sems):
    ssems = sems[:NBD]
    dsems = sems[NBD:]
    c = lax.axis_index("c")
    s = lax.axis_index("s")
    wid = c * NS + s
    cbase = wid * NCHUNK
    _tile_rows_copy(s, lambda r, n: zdeg_hbm.at[pl.ds(r, n)],
                    lambda r, n: acc_s.at[pl.ds(r, n)])
    _tile_rows_copy(s, lambda r, n: zdeg_hbm.at[pl.ds(r, n)],
                    lambda r, n: acc_d.at[pl.ds(r, n)])
    pltpu.sync_copy(src2_hbm.at[pl.ds(cbase, NCHUNK)], sidx)
    pltpu.sync_copy(dst2_hbm.at[pl.ds(cbase, NCHUNK)], didx)
    pltpu.sync_copy(ones_hbm, ones_v)
    plsc.subcore_barrier()

    def group(gi, carry):
        for bb in range(NBD):
            j = gi * NBD + bb

            @pl.when(j >= NBD)
            def _():
                pltpu.make_async_copy(ones_v, acc_s.at[sidx.at[j - NBD]],
                                      ssems[bb]).wait()
                pltpu.make_async_copy(ones_v, acc_d.at[didx.at[j - NBD]],
                                      dsems[bb]).wait()

            pltpu.async_copy(ones_v, acc_s.at[sidx.at[j]], ssems[bb],
                             add=True)
            pltpu.async_copy(ones_v, acc_d.at[didx.at[j]], dsems[bb],
                             add=True)
        return carry

    lax.fori_loop(0, DGROUPS, group, 0)
    for cc in range(NCHUNK - NBD, NCHUNK):
        b = cc % NBD
        pltpu.make_async_copy(ones_v, acc_s.at[sidx.at[cc]], ssems[b]).wait()
        pltpu.make_async_copy(ones_v, acc_d.at[didx.at[cc]], dsems[b]).wait()
    plsc.subcore_barrier()
    _tile_rows_copy(s, lambda r, n: acc_s.at[pl.ds(r, n)],
                    lambda r, n: out_hbm.at[c, 0, pl.ds(r, n)])
    _tile_rows_copy(s, lambda r, n: acc_d.at[pl.ds(r, n)],
                    lambda r, n: out_hbm.at[c, 1, pl.ds(r, n)])


NB = 4
KLAG = 3
AGROUPS = -(-(NCHUNK + KLAG) // NB)


def _agg_body(src2_hbm, dst2_hbm, m_hbm, zeros_hbm, out_hbm,
              sidx, didx, rows, acc, *sems):
    gsems = sems[:NB]
    ssems = sems[NB:]
    c = lax.axis_index("c")
    s = lax.axis_index("s")
    wid = c * NS + s
    cbase = wid * NCHUNK
    _tile_rows_copy(s, lambda r, n: zeros_hbm.at[pl.ds(r, n)],
                    lambda r, n: acc.at[pl.ds(r, n)])
    pltpu.sync_copy(src2_hbm.at[pl.ds(cbase, NCHUNK)], sidx)
    pltpu.sync_copy(dst2_hbm.at[pl.ds(cbase, NCHUNK)], didx)
    plsc.subcore_barrier()

    def group(gi, carry):
        for bb in range(NB):
            j = gi * NB + bb

            @pl.when(j < NCHUNK)
            def _():
                @pl.when(j >= NB)
                def _():
                    pltpu.make_async_copy(rows.at[bb],
                                          acc.at[didx.at[j - NB]],
                                          ssems[bb]).wait()

                pltpu.async_copy(m_hbm.at[sidx.at[j]], rows.at[bb],
                                 gsems[bb])

            @pl.when(jnp.logical_and(j >= KLAG, j < NCHUNK + KLAG))
            def _():
                cs = j - KLAG
                bs = (bb - KLAG) % NB
                pltpu.make_async_copy(m_hbm.at[sidx.at[cs]], rows.at[bs],
                                      gsems[bs]).wait()
                pltpu.async_copy(rows.at[bs], acc.at[didx.at[cs]],
                                 ssems[bs], add=True)
        return carry

    lax.fori_loop(0, AGROUPS, group, 0)
    for cc in range(NCHUNK - NB, NCHUNK):
        b = cc % NB
        pltpu.make_async_copy(rows.at[b], acc.at[didx.at[cc]],
                              ssems[b]).wait()
    plsc.subcore_barrier()
    _tile_rows_copy(s, lambda r, n: acc.at[pl.ds(r, n)],
                    lambda r, n: out_hbm.at[c, pl.ds(r, n)])


@functools.cache
def _sc_kernels():
    mesh = plsc.VectorSubcoreMesh(core_axis_name="c", subcore_axis_name="s")
    deg = pl.kernel(
        _deg_body,
        out_type=jax.ShapeDtypeStruct((NC, 2, N, DW), jnp.float32),
        mesh=mesh,
        compiler_params=pltpu.CompilerParams(use_tc_tiling_on_sc=False),
        scratch_types=[
            pltpu.VMEM((NCHUNK, CH), jnp.int32),
            pltpu.VMEM((NCHUNK, CH), jnp.int32),
            pltpu.VMEM((CH, DW), jnp.float32),
            pltpu.VMEM_SHARED((N, DW), jnp.float32),
            pltpu.VMEM_SHARED((N, DW), jnp.float32),
        ] + [pltpu.SemaphoreType.DMA] * (2 * NBD),
    )
    agg = pl.kernel(
        _agg_body,
        out_type=jax.ShapeDtypeStruct((NC, N, D // 2), jnp.int32),
        mesh=mesh,
        compiler_params=pltpu.CompilerParams(use_tc_tiling_on_sc=False),
        scratch_types=[
            pltpu.VMEM((NCHUNK, CH), jnp.int32),
            pltpu.VMEM((NCHUNK, CH), jnp.int32),
            pltpu.VMEM((NB, CH, D // 2), jnp.int32),
            pltpu.VMEM_SHARED((N, D // 2), jnp.int32),
        ] + [pltpu.SemaphoreType.DMA] * (2 * NB),
    )
    return deg, agg


N2 = N // 2
D2 = 2 * D
QBUDGET = 32766.0


def _bc2(v2, w):
    return jnp.concatenate([jnp.broadcast_to(v2[:, 0:1], (N2, w)),
                            jnp.broadcast_to(v2[:, 1:2], (N2, w))], axis=1)


def _blockdiag(w):
    z = jnp.zeros((D, D), dtype=w.dtype)
    return jnp.concatenate([jnp.concatenate([w, z], axis=1),
                            jnp.concatenate([z, w], axis=1)], axis=0)


def _quantize5(m5, dmax):
    amp = jnp.floor(QBUDGET / jnp.maximum(dmax, 1.0))
    mx = jnp.max(jnp.abs(m5), axis=0)
    mm = jnp.maximum(mx[:D], mx[D:])[None, :]
    inv_scale = jnp.maximum(mm, 1e-30) / (amp - 1.0)
    rs = 1.0 / jnp.concatenate([inv_scale, inv_scale], axis=1)
    q = jnp.round(m5 * rs).astype(jnp.int32)
    ai = amp.astype(jnp.int32)
    packed = jnp.concatenate(
        [(q[:, 64:128] << 16) + q[:, 0:64] + ai,
         (q[:, 192:256] << 16) + q[:, 128:192] + ai], axis=1)
    return packed, inv_scale, amp


def _embed_body(h_ref, w_ref, b_ref, deg5_ref, x_ref, mq_ref, dis_ref,
                dm_ref, qs_ref, dn_ref):
    w2 = _blockdiag(w_ref[...])
    b2 = jnp.concatenate([b_ref[...], b_ref[...]])
    x5 = jnp.dot(h_ref[...], w2, preferred_element_type=jnp.float32) + b2
    de = deg5_ref[...]
    dis = jnp.where(de > 0.0,
                    1.0 / jnp.sqrt(jnp.maximum(de, 1.0)),
                    0.0)
    dmax = jnp.max(de[1])
    m5 = x5 * _bc2(dis[0], D)
    mq, qs, _ = _quantize5(m5, dmax)
    x_ref[...] = x5
    mq_ref[...] = mq
    dis_ref[...] = dis
    dm_ref[...] = jnp.reshape(dmax, (1, 1))
    qs_ref[...] = qs
    dn_ref[...] = de[1]


_embed_call = pl.pallas_call(
    _embed_body,
    compiler_params=pltpu.CompilerParams(vmem_limit_bytes=100 * 1024 * 1024),
    out_shape=[
        jax.ShapeDtypeStruct((N2, D2), jnp.float32),
        jax.ShapeDtypeStruct((N2, D), jnp.int32),
        jax.ShapeDtypeStruct((2, N2, 2), jnp.float32),
        jax.ShapeDtypeStruct((1, 1), jnp.float32),
        jax.ShapeDtypeStruct((1, D), jnp.float32),
        jax.ShapeDtypeStruct((N2, 2), jnp.float32),
    ],
)


def _layer_body(part_ref, x_ref, dis_ref, dm_ref, qs_ref, dn_ref,
                w_ref, b_ref, g_ref, be_ref, y_ref, mq_ref, qs2_ref):
    dmax = dm_ref[0, 0]
    amp = jnp.floor(QBUDGET / jnp.maximum(dmax, 1.0))
    psum = part_ref[0] + part_ref[1]
    low = (psum & 0xFFFF).astype(jnp.float32) - _bc2(dn_ref[...], 64) * amp
    high = (psum >> 16).astype(jnp.float32)
    agg = jnp.concatenate([low[:, 0:64], high[:, 0:64],
                           low[:, 64:128], high[:, 64:128]], axis=1)
    qs = qs_ref[...]
    agg = agg * jnp.concatenate([qs, qs], axis=1) * _bc2(dis_ref[1], D)
    w2 = _blockdiag(w_ref[...])
    out = jnp.dot(agg, w2, preferred_element_type=jnp.float32)
    out = out + jnp.concatenate([b_ref[...], b_ref[...]])
    s1 = jnp.sum(out, axis=0)
    mean = (s1[:D] + s1[D:]) * (1.0 / N)
    cent = out - jnp.concatenate([mean, mean])
    s2 = jnp.sum(cent * cent, axis=0)
    var = (s2[:D] + s2[D:]) * (1.0 / N)
    rstd = lax.rsqrt(var + 1e-5) * g_ref[...]
    out = cent * jnp.concatenate([rstd, rstd]) \
        + jnp.concatenate([be_ref[...], be_ref[...]])
    y5 = jnp.maximum(out, 0.0) + x_ref[...]
    m5 = y5 * _bc2(dis_ref[0], D)
    mq, qs2, _ = _quantize5(m5, dmax)
    y_ref[...] = y5
    mq_ref[...] = mq
    qs2_ref[...] = qs2


_layer_call = pl.pallas_call(
    _layer_body,
    compiler_params=pltpu.CompilerParams(vmem_limit_bytes=100 * 1024 * 1024),
    out_shape=[
        jax.ShapeDtypeStruct((N2, D2), jnp.float32),
        jax.ShapeDtypeStruct((N2, D), jnp.int32),
        jax.ShapeDtypeStruct((1, D), jnp.float32),
    ],
)


def kernel(g, h, e, W_embed, b_embed, Ws, bs, gammas, betas):
    src2 = g[0].reshape(E // CH, CH)
    dst2 = g[1].reshape(E // CH, CH)
    ones = jnp.ones((CH, DW), dtype=jnp.float32)
    zdeg = jnp.zeros((N, DW), dtype=jnp.float32)
    zeros = jnp.zeros((N, D // 2), dtype=jnp.int32)
    deg_kernel, agg_kernel = _sc_kernels()
    degs = deg_kernel(src2, dst2, ones, zdeg)
    deg5 = (degs[0, :, :, 0] + degs[1, :, :, 0]).reshape(2, N2, 2)
    x, mq, dis, dmax, qs, degn = _embed_call(h.reshape(N2, D2), W_embed,
                                             b_embed, deg5)
    for i in range(L):
        part = agg_kernel(src2, dst2, mq.reshape(N, D // 2), zeros)
        x, mq, qs = _layer_call(part.reshape(NC, N2, D), x, dis, dmax,
                                qs, degn, Ws[i], bs[i], gammas[i], betas[i])
    return x.reshape(N, D)
